# baseline (speedup 1.0000x reference)
import functools

import jax
import jax.numpy as jnp
from jax import lax
from jax.experimental import pallas as pl
from jax.experimental.pallas import tpu as pltpu
from jax.experimental.pallas import tpu_sc as plsc

N_NODES = 10000
N_EDGES = 320000
D_FEAT = 128

NC = 2
NS = 16
B = 112
CHUNK = 2 * N_EDGES // (NC * NS)
NBB = 12
NB = -(-CHUNK // (B * NBB)) * NBB
NG = NB // NBB
PAD_CHUNK = NB * B
N_PAD = 10240
ROWS_PER_TILE = N_PAD // NS
RQ = 80


def _sc_kernel(x_hbm, src_hbm, dst_hbm, t_hbm, out_hbm,
               src_v, dst_v, t_v, rows0, rows1, rows2, acc,
               gsem0, gsem1, gsem2, hsem0, hsem1, hsem2,
               ssem0, ssem1, ssem2):
    c = lax.axis_index("c")
    s = lax.axis_index("s")
    w = c * NS + s
    row0 = s * ROWS_PER_TILE
    bufs = ((rows0, gsem0, hsem0, ssem0), (rows1, gsem1, hsem1, ssem1),
            (rows2, gsem2, hsem2, ssem2))
    H = B // 2

    def gather_start(j, rows, semA, semB):
        pltpu.async_copy(x_hbm.at[src_v.at[j, pl.ds(0, H)]],
                         rows.at[pl.ds(0, H)], semA)
        pltpu.async_copy(x_hbm.at[src_v.at[j, pl.ds(H, H)]],
                         rows.at[pl.ds(H, H)], semB)

    def gather_wait(j, rows, semA, semB):
        pltpu.make_async_copy(x_hbm.at[src_v.at[j, pl.ds(0, H)]],
                              rows.at[pl.ds(0, H)], semA).wait()
        pltpu.make_async_copy(x_hbm.at[src_v.at[j, pl.ds(H, H)]],
                              rows.at[pl.ds(H, H)], semB).wait()

    zero16 = jnp.zeros((16,), jnp.float32)

    def zrow(r, carry):
        for k in range(D_FEAT // 16):
            rows0[r, pl.ds(k * 16, 16)] = zero16
        return carry

    lax.fori_loop(0, RQ, zrow, 0)
    for q in range(ROWS_PER_TILE // RQ):
        pltpu.sync_copy(rows0.at[pl.ds(0, RQ)],
                        acc.at[pl.ds(row0 + q * RQ, RQ)])
    plsc.subcore_barrier()

    def scale_rows(rows, j):
        def block(bb, inner):
            t16 = t_v[j, pl.ds(bb * 16, 16)]
            for l in range(16):
                tl = t16[l]
                e = bb * 16 + l
                for k in range(D_FEAT // 16):
                    sl = pl.ds(k * 16, 16)
                    rows[e, sl] = rows[e, sl] * tl
            return inner

        lax.fori_loop(0, B // 16, block, 0)

    def group(g, gcarry):
        @pl.when(g > 0)
        def _():
            pltpu.make_async_copy(rows2, acc.at[dst_v.at[0]], ssem2).wait()

        pltpu.sync_copy(src_hbm.at[w, g], src_v)
        pltpu.sync_copy(dst_hbm.at[w, g], dst_v)
        pltpu.sync_copy(t_hbm.at[w, g], t_v)

        gather_start(0, rows0, gsem0, hsem0)
        gather_start(1, rows1, gsem1, hsem1)

        def triple(p, carry):
            for b in range(3):
                jj = p * 3 + b
                rows, gsem, hsem, _ssem = bufs[b]
                prows, pgsem, phsem, pssem = bufs[(b + 2) % 3]

                gather_wait(jj, rows, gsem, hsem)
                scale_rows(rows, jj)
                pltpu.async_copy(rows, acc.at[dst_v.at[jj]], _ssem, add=True)

                @pl.when(jj >= 1)
                def _():
                    pltpu.make_async_copy(
                        prows, acc.at[dst_v.at[0]], pssem).wait()

                @pl.when(jj + 2 < NBB)
                def _():
                    gather_start(jj + 2, prows, pgsem, phsem)
            return carry

        lax.fori_loop(0, NBB // 3, triple, 0)
        return gcarry

    lax.fori_loop(0, NG, group, 0)

    pltpu.make_async_copy(rows2, acc.at[dst_v.at[0]], ssem2).wait()
    plsc.subcore_barrier()

    for q in range(ROWS_PER_TILE // RQ):
        r = row0 + q * RQ
        pltpu.sync_copy(acc.at[pl.ds(r, RQ)], rows0.at[pl.ds(0, RQ)])
        pltpu.sync_copy(rows0.at[pl.ds(0, RQ)], out_hbm.at[c, pl.ds(r, RQ)])


@jax.jit
def kernel(x0, x1, edge_index0, edge_index1, t0, t1):
    x = jnp.concatenate([x0, x1], axis=0)
    src = jnp.concatenate([edge_index0[0].astype(jnp.int32),
                           edge_index1[0].astype(jnp.int32) + N_NODES])
    dst = jnp.concatenate([edge_index0[1].astype(jnp.int32),
                           edge_index1[1].astype(jnp.int32)])
    t = jnp.concatenate([t0[:, 0], t1[:, 0]])

    pad = NS * PAD_CHUNK - N_EDGES
    pad_src = (jnp.arange(pad, dtype=jnp.int32) * 37) % N_NODES
    pad_dst = (jnp.arange(pad, dtype=jnp.int32) * 53) % N_NODES
    pad_t = jnp.zeros((pad,), jnp.float32)
    src = jnp.concatenate([src[:N_EDGES], pad_src,
                           src[N_EDGES:], pad_src + N_NODES])
    dst = jnp.concatenate([dst[:N_EDGES], pad_dst,
                           dst[N_EDGES:], pad_dst])
    t = jnp.concatenate([t[:N_EDGES], pad_t, t[N_EDGES:], pad_t])

    src = src.reshape(NC * NS, NG, NBB, B)
    dst = dst.reshape(NC * NS, NG, NBB, B)
    t = t.reshape(NC * NS, NG, NBB, B)

    mesh = plsc.VectorSubcoreMesh(core_axis_name="c", subcore_axis_name="s")
    run = pl.kernel(
        _sc_kernel,
        out_type=jax.ShapeDtypeStruct((NC, N_PAD, D_FEAT), jnp.float32),
        mesh=mesh,
        scratch_types=[
            pltpu.VMEM((NBB, B), jnp.int32),
            pltpu.VMEM((NBB, B), jnp.int32),
            pltpu.VMEM((NBB, B), jnp.float32),
            pltpu.VMEM((B, D_FEAT), jnp.float32),
            pltpu.VMEM((B, D_FEAT), jnp.float32),
            pltpu.VMEM((B, D_FEAT), jnp.float32),
            pltpu.VMEM_SHARED((N_PAD, D_FEAT), jnp.float32),
            pltpu.SemaphoreType.DMA,
            pltpu.SemaphoreType.DMA,
            pltpu.SemaphoreType.DMA,
            pltpu.SemaphoreType.DMA,
            pltpu.SemaphoreType.DMA,
            pltpu.SemaphoreType.DMA,
            pltpu.SemaphoreType.DMA,
            pltpu.SemaphoreType.DMA,
            pltpu.SemaphoreType.DMA,
        ],
    )
    out = run(x, src, dst, t)
    return jnp.swapaxes(out[:, :N_NODES, :], 0, 1)


# speedup vs baseline: 1.0006x; 1.0006x over previous
import functools

import jax
import jax.numpy as jnp
from jax import lax
from jax.experimental import pallas as pl
from jax.experimental.pallas import tpu as pltpu
from jax.experimental.pallas import tpu_sc as plsc

N_NODES = 10000
N_EDGES = 320000
D_FEAT = 128

NC = 2
NS = 16
B = 112
CHUNK = 2 * N_EDGES // (NC * NS)
NBB = 12
NB = -(-CHUNK // (B * NBB)) * NBB
NG = NB // NBB
PAD_CHUNK = NB * B
N_PAD = 10240
ROWS_PER_TILE = N_PAD // NS
RQ = 80


def _sc_kernel(x_hbm, src_hbm, dst_hbm, t_hbm, out_hbm,
               src_v, dst_v, t_v, rows0, rows1, rows2, acc,
               gsem0, gsem1, gsem2, hsem0, hsem1, hsem2,
               ssem0, ssem1, ssem2):
    c = lax.axis_index("c")
    s = lax.axis_index("s")
    w = c * NS + s
    row0 = s * ROWS_PER_TILE
    bufs = ((rows0, gsem0, hsem0, ssem0), (rows1, gsem1, hsem1, ssem1),
            (rows2, gsem2, hsem2, ssem2))
    H = B // 2

    def gather_start(j, rows, semA, semB):
        pltpu.async_copy(x_hbm.at[src_v.at[j, pl.ds(0, H)]],
                         rows.at[pl.ds(0, H)], semA)
        pltpu.async_copy(x_hbm.at[src_v.at[j, pl.ds(H, H)]],
                         rows.at[pl.ds(H, H)], semB)

    def gather_wait(j, rows, semA, semB):
        pltpu.make_async_copy(x_hbm.at[src_v.at[j, pl.ds(0, H)]],
                              rows.at[pl.ds(0, H)], semA).wait()
        pltpu.make_async_copy(x_hbm.at[src_v.at[j, pl.ds(H, H)]],
                              rows.at[pl.ds(H, H)], semB).wait()

    zero16 = jnp.zeros((16,), jnp.float32)

    def zrow(r, carry):
        for k in range(D_FEAT // 16):
            rows0[r, pl.ds(k * 16, 16)] = zero16
        return carry

    lax.fori_loop(0, RQ, zrow, 0)
    for q in range(ROWS_PER_TILE // RQ):
        pltpu.sync_copy(rows0.at[pl.ds(0, RQ)],
                        acc.at[pl.ds(row0 + q * RQ, RQ)])
    plsc.subcore_barrier()

    def scale_rows(rows, j):
        def block(bb, inner):
            t16 = t_v[j, pl.ds(bb * 16, 16)]
            for l in range(16):
                tl = t16[l]
                e = bb * 16 + l
                for k in range(D_FEAT // 16):
                    sl = pl.ds(k * 16, 16)
                    rows[e, sl] = rows[e, sl] * tl
            return inner

        lax.fori_loop(0, B // 16, block, 0)

    def group(g, gcarry):
        @pl.when(g > 0)
        def _():
            pltpu.make_async_copy(rows1, acc.at[dst_v.at[0]], ssem1).wait()
            pltpu.make_async_copy(rows2, acc.at[dst_v.at[0]], ssem2).wait()

        pltpu.sync_copy(src_hbm.at[w, g], src_v)
        pltpu.sync_copy(dst_hbm.at[w, g], dst_v)
        pltpu.sync_copy(t_hbm.at[w, g], t_v)

        gather_start(0, rows0, gsem0, hsem0)

        def triple(p, carry):
            for b in range(3):
                jj = p * 3 + b
                rows, gsem, hsem, _ssem = bufs[b]
                nrows, ngsem, nhsem, nssem = bufs[(b + 1) % 3]

                @pl.when(jj >= 2)
                def _():
                    pltpu.make_async_copy(
                        nrows, acc.at[dst_v.at[0]], nssem).wait()

                @pl.when(jj + 1 < NBB)
                def _():
                    gather_start(jj + 1, nrows, ngsem, nhsem)

                gather_wait(jj, rows, gsem, hsem)
                scale_rows(rows, jj)
                pltpu.async_copy(rows, acc.at[dst_v.at[jj]], _ssem, add=True)
            return carry

        lax.fori_loop(0, NBB // 3, triple, 0)
        return gcarry

    lax.fori_loop(0, NG, group, 0)

    pltpu.make_async_copy(rows1, acc.at[dst_v.at[0]], ssem1).wait()
    pltpu.make_async_copy(rows2, acc.at[dst_v.at[0]], ssem2).wait()
    plsc.subcore_barrier()

    for q in range(ROWS_PER_TILE // RQ):
        r = row0 + q * RQ
        pltpu.sync_copy(acc.at[pl.ds(r, RQ)], rows0.at[pl.ds(0, RQ)])
        pltpu.sync_copy(rows0.at[pl.ds(0, RQ)], out_hbm.at[c, pl.ds(r, RQ)])


@jax.jit
def kernel(x0, x1, edge_index0, edge_index1, t0, t1):
    x = jnp.concatenate([x0, x1], axis=0)
    src = jnp.concatenate([edge_index0[0].astype(jnp.int32),
                           edge_index1[0].astype(jnp.int32) + N_NODES])
    dst = jnp.concatenate([edge_index0[1].astype(jnp.int32),
                           edge_index1[1].astype(jnp.int32)])
    t = jnp.concatenate([t0[:, 0], t1[:, 0]])

    pad = NS * PAD_CHUNK - N_EDGES
    pad_src = (jnp.arange(pad, dtype=jnp.int32) * 37) % N_NODES
    pad_dst = (jnp.arange(pad, dtype=jnp.int32) * 53) % N_NODES
    pad_t = jnp.zeros((pad,), jnp.float32)
    src = jnp.concatenate([src[:N_EDGES], pad_src,
                           src[N_EDGES:], pad_src + N_NODES])
    dst = jnp.concatenate([dst[:N_EDGES], pad_dst,
                           dst[N_EDGES:], pad_dst])
    t = jnp.concatenate([t[:N_EDGES], pad_t, t[N_EDGES:], pad_t])

    src = src.reshape(NC * NS, NG, NBB, B)
    dst = dst.reshape(NC * NS, NG, NBB, B)
    t = t.reshape(NC * NS, NG, NBB, B)

    mesh = plsc.VectorSubcoreMesh(core_axis_name="c", subcore_axis_name="s")
    run = pl.kernel(
        _sc_kernel,
        out_type=jax.ShapeDtypeStruct((NC, N_PAD, D_FEAT), jnp.float32),
        mesh=mesh,
        scratch_types=[
            pltpu.VMEM((NBB, B), jnp.int32),
            pltpu.VMEM((NBB, B), jnp.int32),
            pltpu.VMEM((NBB, B), jnp.float32),
            pltpu.VMEM((B, D_FEAT), jnp.float32),
            pltpu.VMEM((B, D_FEAT), jnp.float32),
            pltpu.VMEM((B, D_FEAT), jnp.float32),
            pltpu.VMEM_SHARED((N_PAD, D_FEAT), jnp.float32),
            pltpu.SemaphoreType.DMA,
            pltpu.SemaphoreType.DMA,
            pltpu.SemaphoreType.DMA,
            pltpu.SemaphoreType.DMA,
            pltpu.SemaphoreType.DMA,
            pltpu.SemaphoreType.DMA,
            pltpu.SemaphoreType.DMA,
            pltpu.SemaphoreType.DMA,
            pltpu.SemaphoreType.DMA,
        ],
    )
    out = run(x, src, dst, t)
    return jnp.swapaxes(out[:, :N_NODES, :], 0, 1)


# speedup vs baseline: 1.0725x; 1.0718x over previous
import functools

import jax
import jax.numpy as jnp
from jax import lax
from jax.experimental import pallas as pl
from jax.experimental.pallas import tpu as pltpu
from jax.experimental.pallas import tpu_sc as plsc

N_NODES = 10000
N_EDGES = 320000
D_FEAT = 128

NC = 2
NS = 16
B = 96
CHUNK = 2 * N_EDGES // (NC * NS)
NBB = 15
NB = -(-CHUNK // (B * NBB)) * NBB
NG = NB // NBB
PAD_CHUNK = NB * B
N_PAD = 10240
ROWS_PER_TILE = N_PAD // NS
RQ = 80


def _sc_kernel(x_hbm, src_hbm, dst_hbm, t_hbm, out_hbm,
               src_va, src_vb, dst_v, t_v, rows0, rows1, rows2, acc,
               gsem0, gsem1, gsem2, hsem0, hsem1, hsem2,
               ssem0, ssem1, ssem2, slsema, slsemb):
    c = lax.axis_index("c")
    s = lax.axis_index("s")
    w = c * NS + s
    row0 = s * ROWS_PER_TILE
    bufs = ((rows0, gsem0, hsem0, ssem0), (rows1, gsem1, hsem1, ssem1),
            (rows2, gsem2, hsem2, ssem2))
    H = B // 2

    def gather_start(src_v, j, rows, semA, semB):
        pltpu.async_copy(x_hbm.at[src_v.at[j, pl.ds(0, H)]],
                         rows.at[pl.ds(0, H)], semA)
        pltpu.async_copy(x_hbm.at[src_v.at[j, pl.ds(H, H)]],
                         rows.at[pl.ds(H, H)], semB)

    def gather_wait(src_v, j, rows, semA, semB):
        pltpu.make_async_copy(x_hbm.at[src_v.at[j, pl.ds(0, H)]],
                              rows.at[pl.ds(0, H)], semA).wait()
        pltpu.make_async_copy(x_hbm.at[src_v.at[j, pl.ds(H, H)]],
                              rows.at[pl.ds(H, H)], semB).wait()

    zero16 = jnp.zeros((16,), jnp.float32)

    def zrow(r, carry):
        for k in range(D_FEAT // 16):
            rows0[r, pl.ds(k * 16, 16)] = zero16
        return carry

    lax.fori_loop(0, RQ, zrow, 0)
    for q in range(ROWS_PER_TILE // RQ):
        pltpu.sync_copy(rows0.at[pl.ds(0, RQ)],
                        acc.at[pl.ds(row0 + q * RQ, RQ)])
    plsc.subcore_barrier()

    def scale_rows(rows, j):
        def block(bb, inner):
            t16 = t_v[j, pl.ds(bb * 16, 16)]
            for l in range(16):
                tl = t16[l]
                e = bb * 16 + l
                for k in range(D_FEAT // 16):
                    sl = pl.ds(k * 16, 16)
                    rows[e, sl] = rows[e, sl] * tl
            return inner

        lax.fori_loop(0, B // 16, block, 0)

    def group_body(g, cur_src, nxt_src, nxt_sem):
        first = g == 0
        has_next = g < NG - 1

        @pl.when(jnp.logical_not(first))
        def _():
            pltpu.make_async_copy(rows1, acc.at[dst_v.at[0]], ssem1).wait()
            pltpu.make_async_copy(rows2, acc.at[dst_v.at[0]], ssem2).wait()

        @pl.when(first)
        def _():
            pltpu.sync_copy(src_hbm.at[w, g], cur_src)
            gather_start(cur_src, 0, rows0, gsem0, hsem0)

        pltpu.sync_copy(dst_hbm.at[w, g], dst_v)
        pltpu.sync_copy(t_hbm.at[w, g], t_v)

        @pl.when(has_next)
        def _():
            pltpu.async_copy(src_hbm.at[w, g + 1], nxt_src, nxt_sem)

        def triple(p, carry):
            for b in range(3):
                jj = p * 3 + b
                rows, gsem, hsem, _ssem = bufs[b]
                nrows, ngsem, nhsem, nssem = bufs[(b + 1) % 3]

                @pl.when(jj >= 2)
                def _():
                    pltpu.make_async_copy(
                        nrows, acc.at[dst_v.at[0]], nssem).wait()

                @pl.when(jj + 1 < NBB)
                def _():
                    gather_start(cur_src, jj + 1, nrows, ngsem, nhsem)

                @pl.when(jnp.logical_and(jj == NBB - 1, has_next))
                def _():
                    pltpu.make_async_copy(
                        src_hbm.at[w, g + 1], nxt_src, nxt_sem).wait()
                    gather_start(nxt_src, 0, nrows, ngsem, nhsem)

                gather_wait(cur_src, jj, rows, gsem, hsem)
                scale_rows(rows, jj)
                pltpu.async_copy(rows, acc.at[dst_v.at[jj]], _ssem, add=True)
            return carry

        lax.fori_loop(0, NBB // 3, triple, 0)

    def dgroup(gg, gcarry):
        group_body(2 * gg, src_va, src_vb, slsemb)
        group_body(2 * gg + 1, src_vb, src_va, slsema)
        return gcarry

    lax.fori_loop(0, NG // 2, dgroup, 0)

    pltpu.make_async_copy(rows1, acc.at[dst_v.at[0]], ssem1).wait()
    pltpu.make_async_copy(rows2, acc.at[dst_v.at[0]], ssem2).wait()
    plsc.subcore_barrier()

    for q in range(ROWS_PER_TILE // RQ):
        r = row0 + q * RQ
        pltpu.sync_copy(acc.at[pl.ds(r, RQ)], rows0.at[pl.ds(0, RQ)])
        pltpu.sync_copy(rows0.at[pl.ds(0, RQ)], out_hbm.at[c, pl.ds(r, RQ)])


@jax.jit
def kernel(x0, x1, edge_index0, edge_index1, t0, t1):
    x = jnp.concatenate([x0, x1], axis=0)
    src = jnp.concatenate([edge_index0[0].astype(jnp.int32),
                           edge_index1[0].astype(jnp.int32) + N_NODES])
    dst = jnp.concatenate([edge_index0[1].astype(jnp.int32),
                           edge_index1[1].astype(jnp.int32)])
    t = jnp.concatenate([t0[:, 0], t1[:, 0]])

    pad = NS * PAD_CHUNK - N_EDGES
    pad_src = (jnp.arange(pad, dtype=jnp.int32) * 37) % N_NODES
    pad_dst = (jnp.arange(pad, dtype=jnp.int32) * 53) % N_NODES
    pad_t = jnp.zeros((pad,), jnp.float32)
    src = jnp.concatenate([src[:N_EDGES], pad_src,
                           src[N_EDGES:], pad_src + N_NODES])
    dst = jnp.concatenate([dst[:N_EDGES], pad_dst,
                           dst[N_EDGES:], pad_dst])
    t = jnp.concatenate([t[:N_EDGES], pad_t, t[N_EDGES:], pad_t])

    src = src.reshape(NC * NS, NG, NBB, B)
    dst = dst.reshape(NC * NS, NG, NBB, B)
    t = t.reshape(NC * NS, NG, NBB, B)

    mesh = plsc.VectorSubcoreMesh(core_axis_name="c", subcore_axis_name="s")
    run = pl.kernel(
        _sc_kernel,
        out_type=jax.ShapeDtypeStruct((NC, N_PAD, D_FEAT), jnp.float32),
        mesh=mesh,
        scratch_types=[
            pltpu.VMEM((NBB, B), jnp.int32),
            pltpu.VMEM((NBB, B), jnp.int32),
            pltpu.VMEM((NBB, B), jnp.int32),
            pltpu.VMEM((NBB, B), jnp.float32),
            pltpu.VMEM((B, D_FEAT), jnp.float32),
            pltpu.VMEM((B, D_FEAT), jnp.float32),
            pltpu.VMEM((B, D_FEAT), jnp.float32),
            pltpu.VMEM_SHARED((N_PAD, D_FEAT), jnp.float32),
            pltpu.SemaphoreType.DMA,
            pltpu.SemaphoreType.DMA,
            pltpu.SemaphoreType.DMA,
            pltpu.SemaphoreType.DMA,
            pltpu.SemaphoreType.DMA,
            pltpu.SemaphoreType.DMA,
            pltpu.SemaphoreType.DMA,
            pltpu.SemaphoreType.DMA,
            pltpu.SemaphoreType.DMA,
            pltpu.SemaphoreType.DMA,
            pltpu.SemaphoreType.DMA,
        ],
    )
    out = run(x, src, dst, t)
    return jnp.swapaxes(out[:, :N_NODES, :], 0, 1)


# speedup vs baseline: 1.1014x; 1.0269x over previous
import functools

import jax
import jax.numpy as jnp
from jax import lax
from jax.experimental import pallas as pl
from jax.experimental.pallas import tpu as pltpu
from jax.experimental.pallas import tpu_sc as plsc

N_NODES = 10000
N_EDGES = 320000
D_FEAT = 128

NC = 2
NS = 16
B = 96
CHUNK = 2 * N_EDGES // (NC * NS)
NBB = 15
NB = -(-CHUNK // (B * NBB)) * NBB
NG = NB // NBB
PAD_CHUNK = NB * B
N_PAD = 10240
ROWS_PER_TILE = N_PAD // NS
RQ = 80


def _sc_kernel(x_hbm, src_hbm, dst_hbm, t_hbm, out_hbm,
               src_va, src_vb, dst_v, t_v, rows0, rows1, rows2, acc,
               gsem0, gsem1, gsem2, hsem0, hsem1, hsem2,
               ssem0, ssem1, ssem2, slsema, slsemb, dtsem):
    c = lax.axis_index("c")
    s = lax.axis_index("s")
    w = c * NS + s
    row0 = s * ROWS_PER_TILE
    bufs = ((rows0, gsem0, hsem0, ssem0), (rows1, gsem1, hsem1, ssem1),
            (rows2, gsem2, hsem2, ssem2))
    H = B // 2

    def gather_start(src_v, j, rows, semA, semB):
        pltpu.async_copy(x_hbm.at[src_v.at[j, pl.ds(0, H)]],
                         rows.at[pl.ds(0, H)], semA)
        pltpu.async_copy(x_hbm.at[src_v.at[j, pl.ds(H, H)]],
                         rows.at[pl.ds(H, H)], semB)

    def gather_wait(src_v, j, rows, semA, semB):
        pltpu.make_async_copy(x_hbm.at[src_v.at[j, pl.ds(0, H)]],
                              rows.at[pl.ds(0, H)], semA).wait()
        pltpu.make_async_copy(x_hbm.at[src_v.at[j, pl.ds(H, H)]],
                              rows.at[pl.ds(H, H)], semB).wait()

    zero16 = jnp.zeros((16,), jnp.float32)

    def zrow(r, carry):
        for k in range(D_FEAT // 16):
            rows0[r, pl.ds(k * 16, 16)] = zero16
        return carry

    lax.fori_loop(0, RQ, zrow, 0)
    for q in range(ROWS_PER_TILE // RQ):
        pltpu.sync_copy(rows0.at[pl.ds(0, RQ)],
                        acc.at[pl.ds(row0 + q * RQ, RQ)])
    plsc.subcore_barrier()

    def scale_rows(rows, j):
        def block(bb, inner):
            t16 = t_v[j, pl.ds(bb * 16, 16)]
            for l in range(16):
                tl = t16[l]
                e = bb * 16 + l
                for k in range(D_FEAT // 16):
                    sl = pl.ds(k * 16, 16)
                    rows[e, sl] = rows[e, sl] * tl
            return inner

        lax.fori_loop(0, B // 16, block, 0)

    def group_body(g, cur_src, nxt_src, nxt_sem):
        first = g == 0
        has_next = g < NG - 1

        @pl.when(jnp.logical_not(first))
        def _():
            pltpu.make_async_copy(rows1, acc.at[dst_v.at[0]], ssem1).wait()
            pltpu.make_async_copy(rows2, acc.at[dst_v.at[0]], ssem2).wait()

        @pl.when(first)
        def _():
            pltpu.sync_copy(src_hbm.at[w, g], cur_src)
            gather_start(cur_src, 0, rows0, gsem0, hsem0)

        pltpu.async_copy(dst_hbm.at[w, g], dst_v, dtsem)
        pltpu.async_copy(t_hbm.at[w, g], t_v, dtsem)

        @pl.when(has_next)
        def _():
            pltpu.async_copy(src_hbm.at[w, g + 1], nxt_src, nxt_sem)

        def triple(p, carry):
            for b in range(3):
                jj = p * 3 + b
                rows, gsem, hsem, _ssem = bufs[b]
                nrows, ngsem, nhsem, nssem = bufs[(b + 1) % 3]

                @pl.when(jj >= 2)
                def _():
                    pltpu.make_async_copy(
                        nrows, acc.at[dst_v.at[0]], nssem).wait()

                @pl.when(jj + 1 < NBB)
                def _():
                    gather_start(cur_src, jj + 1, nrows, ngsem, nhsem)

                @pl.when(jnp.logical_and(jj == NBB - 1, has_next))
                def _():
                    pltpu.make_async_copy(
                        src_hbm.at[w, g + 1], nxt_src, nxt_sem).wait()
                    gather_start(nxt_src, 0, nrows, ngsem, nhsem)

                gather_wait(cur_src, jj, rows, gsem, hsem)

                @pl.when(jj == 0)
                def _():
                    pltpu.make_async_copy(
                        dst_hbm.at[w, g], dst_v, dtsem).wait()
                    pltpu.make_async_copy(
                        t_hbm.at[w, g], t_v, dtsem).wait()
                scale_rows(rows, jj)
                pltpu.async_copy(rows, acc.at[dst_v.at[jj]], _ssem, add=True)
            return carry

        lax.fori_loop(0, NBB // 3, triple, 0)

    def dgroup(gg, gcarry):
        group_body(2 * gg, src_va, src_vb, slsemb)
        group_body(2 * gg + 1, src_vb, src_va, slsema)
        return gcarry

    lax.fori_loop(0, NG // 2, dgroup, 0)

    pltpu.make_async_copy(rows1, acc.at[dst_v.at[0]], ssem1).wait()
    pltpu.make_async_copy(rows2, acc.at[dst_v.at[0]], ssem2).wait()
    plsc.subcore_barrier()

    for q in range(ROWS_PER_TILE // RQ):
        r = row0 + q * RQ
        pltpu.sync_copy(acc.at[pl.ds(r, RQ)], rows0.at[pl.ds(0, RQ)])
        pltpu.sync_copy(rows0.at[pl.ds(0, RQ)], out_hbm.at[c, pl.ds(r, RQ)])


@jax.jit
def kernel(x0, x1, edge_index0, edge_index1, t0, t1):
    x = jnp.concatenate([x0, x1], axis=0)
    src = jnp.concatenate([edge_index0[0].astype(jnp.int32),
                           edge_index1[0].astype(jnp.int32) + N_NODES])
    dst = jnp.concatenate([edge_index0[1].astype(jnp.int32),
                           edge_index1[1].astype(jnp.int32)])
    t = jnp.concatenate([t0[:, 0], t1[:, 0]])

    pad = NS * PAD_CHUNK - N_EDGES
    pad_src = (jnp.arange(pad, dtype=jnp.int32) * 37) % N_NODES
    pad_dst = (jnp.arange(pad, dtype=jnp.int32) * 53) % N_NODES
    pad_t = jnp.zeros((pad,), jnp.float32)
    src = jnp.concatenate([src[:N_EDGES], pad_src,
                           src[N_EDGES:], pad_src + N_NODES])
    dst = jnp.concatenate([dst[:N_EDGES], pad_dst,
                           dst[N_EDGES:], pad_dst])
    t = jnp.concatenate([t[:N_EDGES], pad_t, t[N_EDGES:], pad_t])

    src = src.reshape(NC * NS, NG, NBB, B)
    dst = dst.reshape(NC * NS, NG, NBB, B)
    t = t.reshape(NC * NS, NG, NBB, B)

    mesh = plsc.VectorSubcoreMesh(core_axis_name="c", subcore_axis_name="s")
    run = pl.kernel(
        _sc_kernel,
        out_type=jax.ShapeDtypeStruct((NC, N_PAD, D_FEAT), jnp.float32),
        mesh=mesh,
        scratch_types=[
            pltpu.VMEM((NBB, B), jnp.int32),
            pltpu.VMEM((NBB, B), jnp.int32),
            pltpu.VMEM((NBB, B), jnp.int32),
            pltpu.VMEM((NBB, B), jnp.float32),
            pltpu.VMEM((B, D_FEAT), jnp.float32),
            pltpu.VMEM((B, D_FEAT), jnp.float32),
            pltpu.VMEM((B, D_FEAT), jnp.float32),
            pltpu.VMEM_SHARED((N_PAD, D_FEAT), jnp.float32),
            pltpu.SemaphoreType.DMA,
            pltpu.SemaphoreType.DMA,
            pltpu.SemaphoreType.DMA,
            pltpu.SemaphoreType.DMA,
            pltpu.SemaphoreType.DMA,
            pltpu.SemaphoreType.DMA,
            pltpu.SemaphoreType.DMA,
            pltpu.SemaphoreType.DMA,
            pltpu.SemaphoreType.DMA,
            pltpu.SemaphoreType.DMA,
            pltpu.SemaphoreType.DMA,
            pltpu.SemaphoreType.DMA,
        ],
    )
    out = run(x, src, dst, t)
    return jnp.swapaxes(out[:, :N_NODES, :], 0, 1)
